# baseline (speedup 1.0000x reference)
import functools

import jax
import jax.numpy as jnp
from jax import lax
from jax.experimental import pallas as pl
from jax.experimental.pallas import tpu as pltpu
from jax.experimental.pallas import tpu_sc as plsc

N = 10000
E = 320000
D = 128
NC = 2
NS = 16
NW = NC * NS
EPW = E // NW
KD = 80
NCHUNKD = EPW // KD
GRPD = 25
NGRPD = NCHUNKD // GRPD
K = 40
NCHUNK = EPW // K
NBUF = 5
GRP = 25
NGRP = NCHUNK // GRP
RPG = GRP // NBUF
NPAD = 10240
RPT = NPAD // NS
PPT = NPAD // NS

def _deg_body(ei_hbm, out_hbm, idx_v, ones_v, stage_v, acc_sh, dsem):
    cid = lax.axis_index("c")
    sid = lax.axis_index("s")
    wid = cid * NS + sid

    zeros16 = jnp.zeros((16,), jnp.float32)
    for j in range(PPT // 16):
        stage_v[pl.ds(j * 16, 16)] = zeros16
    ones16 = jnp.ones((16,), jnp.float32)
    for off in (0, 16, 24):
        ones_v[pl.ds(off, 16)] = ones16

    pltpu.sync_copy(stage_v, acc_sh.at[pl.ds(sid * PPT, PPT)])
    plsc.subcore_barrier()

    def dgroup(g, carry):
        pltpu.sync_copy(ei_hbm.at[1, wid, g], idx_v)

        def fire(j, c2):
            pltpu.async_copy(ones_v, acc_sh.at[idx_v.at[j]], dsem, add=True)
            return c2

        lax.fori_loop(0, GRP, fire, 0)

        def drain(j, c2):
            pltpu.make_async_copy(ones_v, acc_sh.at[idx_v.at[j]],
                                  dsem).wait()
            return c2

        lax.fori_loop(0, GRP, drain, 0)
        return carry

    lax.fori_loop(0, NGRP, dgroup, 0)
    plsc.subcore_barrier()

    pltpu.sync_copy(acc_sh.at[pl.ds(sid * PPT, PPT)],
                    out_hbm.at[cid, pl.ds(sid * PPT, PPT)])


def _edge_body(g_hbm, ei_hbm, out_hbm,
               src_g, dst_g, r0, r1, r2, r3, r4, zb, acc_sh,
               g0, g1, g2, g3, g4, s0, s1, s2, s3, s4, isem):
    rows = (r0, r1, r2, r3, r4)
    gsem = (g0, g1, g2, g3, g4)
    ssem = (s0, s1, s2, s3, s4)
    cid = lax.axis_index("c")
    sid = lax.axis_index("s")
    wid = cid * NS + sid
    base = sid * RPT

    pltpu.sync_copy(ei_hbm.at[0, wid, 0], src_g.at[0])
    pltpu.sync_copy(ei_hbm.at[1, wid, 0], dst_g.at[0])
    pltpu.async_copy(ei_hbm.at[0, wid, 1], src_g.at[1], isem)
    pltpu.async_copy(ei_hbm.at[1, wid, 1], dst_g.at[1], isem)
    for sl in range(NBUF):
        pltpu.async_copy(g_hbm.at[src_g.at[0, sl]], rows[sl], gsem[sl])

    zeros16 = jnp.zeros((16,), jnp.float32)
    ZR = 32

    def zrow(i, carry):
        for jj in range(D // 16):
            zb[i, pl.ds(jj * 16, 16)] = zeros16
        return carry

    lax.fori_loop(0, ZR, zrow, 0)
    for t in range(RPT // ZR):
        pltpu.sync_copy(zb, acc_sh.at[pl.ds(base + t * ZR, ZR)])
    plsc.subcore_barrier()

    def ground(lr, g):
        p = g % 2
        pn = 1 - p
        last_g = g == NGRP - 1

        @pl.when(jnp.logical_and(lr == 1, jnp.logical_not(last_g)))
        def _():
            pltpu.async_copy(ei_hbm.at[0, wid, g + 1], src_g.at[pn], isem)
            pltpu.async_copy(ei_hbm.at[1, wid, g + 1], dst_g.at[pn], isem)

        @pl.when(jnp.logical_and(lr == RPG - 1, jnp.logical_not(last_g)))
        def _():
            pltpu.make_async_copy(ei_hbm.at[0, wid, g + 1], src_g.at[pn],
                                  isem).wait()
            pltpu.make_async_copy(ei_hbm.at[1, wid, g + 1], dst_g.at[pn],
                                  isem).wait()

        for sl in range(NBUF):
            lc = lr * NBUF + sl
            pltpu.make_async_copy(g_hbm.at[src_g.at[p, lc]], rows[sl],
                                  gsem[sl]).wait()
            pltpu.async_copy(rows[sl], acc_sh.at[dst_g.at[p, lc]],
                             ssem[sl], add=True)

        in_group = lr < RPG - 1
        to_next_group = jnp.logical_and(lr == RPG - 1,
                                        jnp.logical_not(last_g))
        for sl in range(NBUF):
            lc = lr * NBUF + sl

            @pl.when(jnp.logical_or(in_group, to_next_group))
            def _():
                pltpu.make_async_copy(rows[sl], acc_sh.at[dst_g.at[p, lc]],
                                      ssem[sl]).wait()

            @pl.when(in_group)
            def _():
                nlc = (lr + 1) * NBUF + sl
                pltpu.async_copy(g_hbm.at[src_g.at[p, nlc]], rows[sl],
                                 gsem[sl])

            @pl.when(to_next_group)
            def _():
                pltpu.async_copy(g_hbm.at[src_g.at[pn, sl]], rows[sl],
                                 gsem[sl])

    def group(g, carry):
        def rbody(lr, c2):
            ground(lr, g)
            return c2
        lax.fori_loop(0, RPG, rbody, 0)
        return carry

    lax.fori_loop(0, NGRP, group, 0)

    pl_last = (NGRP - 1) % 2
    for sl in range(NBUF):
        lc = (RPG - 1) * NBUF + sl
        pltpu.make_async_copy(rows[sl], acc_sh.at[dst_g.at[pl_last, lc]],
                              ssem[sl]).wait()

    plsc.subcore_barrier()
    sp = pl.ds(base, RPT)
    pltpu.sync_copy(acc_sh.at[sp], out_hbm.at[cid, sp])


@functools.lru_cache(maxsize=None)
def _sc_kernels():
    mesh = plsc.VectorSubcoreMesh(core_axis_name="c", subcore_axis_name="s",
                                  num_cores=NC, num_subcores=NS)
    deg_kernel = pl.kernel(
        _deg_body,
        out_type=jax.ShapeDtypeStruct((NC, NPAD), jnp.float32),
        mesh=mesh,
        scratch_types=[
            pltpu.VMEM((GRP, K), jnp.int32),
            pltpu.VMEM((K,), jnp.float32),
            pltpu.VMEM((PPT,), jnp.float32),
            pltpu.VMEM_SHARED((NPAD,), jnp.float32),
            pltpu.SemaphoreType.DMA,
        ],
    )
    edge_kernel = pl.kernel(
        _edge_body,
        out_type=jax.ShapeDtypeStruct((NC, NPAD, D), jnp.float32),
        mesh=mesh,
        scratch_types=(
            [pltpu.VMEM((2, GRP, K), jnp.int32)] * 2
            + [pltpu.VMEM((K, D), jnp.float32)] * NBUF
            + [pltpu.VMEM((32, D), jnp.float32)]
            + [pltpu.VMEM_SHARED((NPAD, D), jnp.float32)]
            + [pltpu.SemaphoreType.DMA] * (2 * NBUF + 1)
        ),
    )
    return deg_kernel, edge_kernel


def _dis(degp_ref):
    deg = 1.0 + degp_ref[:, 0:1] + degp_ref[:, 1:2]
    return lax.rsqrt(deg)


def _tc_mm_body(x_ref, w_ref, h_ref):
    h_ref[...] = lax.dot_general(x_ref[...], w_ref[...],
                                 (((1,), (0,)), ((), ())),
                                 preferred_element_type=jnp.float32)


_tc_mm = pl.pallas_call(
    _tc_mm_body,
    out_shape=jax.ShapeDtypeStruct((N, D), jnp.float32),
)


def _tc_scale_body(degp_ref, h_ref, g_ref):
    g_ref[...] = h_ref[...] * _dis(degp_ref)


_tc_scale = pl.pallas_call(
    _tc_scale_body,
    out_shape=jax.ShapeDtypeStruct((N, D), jnp.float32),
)


def _tc_mid_body(s_ref, g_ref, degp_ref, b_ref, w_ref, g2_ref):
    dis = _dis(degp_ref)
    u = (s_ref[0, 0:N, :] + s_ref[1, 0:N, :] + g_ref[...]) * dis + b_ref[...]
    r = jnp.maximum(u, 0.0)
    h2 = lax.dot_general(r, w_ref[...], (((1,), (0,)), ((), ())),
                         preferred_element_type=jnp.float32)
    g2_ref[...] = h2 * dis


_tc_mid = pl.pallas_call(
    _tc_mid_body,
    out_shape=jax.ShapeDtypeStruct((N, D), jnp.float32),
)


def _tc_out_body(s_ref, g2_ref, degp_ref, b_ref, out_ref):
    out_ref[...] = (s_ref[0, 0:N, :] + s_ref[1, 0:N, :] + g2_ref[...]) \
        * _dis(degp_ref) + b_ref[...]


_tc_out = pl.pallas_call(
    _tc_out_body,
    out_shape=jax.ShapeDtypeStruct((N, D), jnp.float32),
)


def kernel(x, edge_index, W1, b1, W2, b2):
    ei = edge_index.astype(jnp.int32).reshape(2, NW, NGRP, GRP, K)
    _deg_kernel, _edge_kernel = _sc_kernels()

    h1 = _tc_mm(x, W1)
    degp = _deg_kernel(ei)
    degp_t = degp[:, :N].T

    g1 = _tc_scale(degp_t, h1)
    s1 = _edge_kernel(g1, ei)
    g2 = _tc_mid(s1, g1, degp_t, b1.reshape(1, D), W2)
    s2 = _edge_kernel(g2, ei)
    return _tc_out(s2, g2, degp_t, b2.reshape(1, D))


# speedup vs baseline: 1.0056x; 1.0056x over previous
import functools

import jax
import jax.numpy as jnp
from jax import lax
from jax.experimental import pallas as pl
from jax.experimental.pallas import tpu as pltpu
from jax.experimental.pallas import tpu_sc as plsc

N = 10000
E = 320000
D = 128
NC = 2
NS = 16
NW = NC * NS
EPW = E // NW
KD = 80
NCHUNKD = EPW // KD
GRPD = 25
NGRPD = NCHUNKD // GRPD
K = 40
NCHUNK = EPW // K
NBUF = 5
GRP = 25
NGRP = NCHUNK // GRP
RPG = GRP // NBUF
NPAD = 10240
RPT = NPAD // NS
PPT = NPAD // NS

def _deg_body(ei_hbm, out_hbm, idx_v, ones_v, stage_v, acc_sh, dsem):
    cid = lax.axis_index("c")
    sid = lax.axis_index("s")
    wid = cid * NS + sid

    zeros16 = jnp.zeros((16,), jnp.float32)
    for j in range(PPT // 16):
        stage_v[pl.ds(j * 16, 16)] = zeros16
    ones16 = jnp.ones((16,), jnp.float32)
    for off in (0, 16, 24):
        ones_v[pl.ds(off, 16)] = ones16

    pltpu.sync_copy(stage_v, acc_sh.at[pl.ds(sid * PPT, PPT)])
    plsc.subcore_barrier()

    def dgroup(g, carry):
        pltpu.sync_copy(ei_hbm.at[1, wid, g], idx_v)

        def fire(j, c2):
            pltpu.async_copy(ones_v, acc_sh.at[idx_v.at[j]], dsem, add=True)
            return c2

        lax.fori_loop(0, GRP, fire, 0)

        def drain(j, c2):
            pltpu.make_async_copy(ones_v, acc_sh.at[idx_v.at[j]],
                                  dsem).wait()
            return c2

        lax.fori_loop(0, GRP, drain, 0)
        return carry

    lax.fori_loop(0, NGRP, dgroup, 0)
    plsc.subcore_barrier()

    pltpu.sync_copy(acc_sh.at[pl.ds(sid * PPT, PPT)],
                    out_hbm.at[cid, pl.ds(sid * PPT, PPT)])


def _edge_body(g_hbm, ei_hbm, out_hbm,
               src_g, dst_g, r0, r1, r2, r3, r4, zb, acc_sh,
               g0, g1, g2, g3, g4, s0, s1, s2, s3, s4, isem):
    rows = (r0, r1, r2, r3, r4)
    gsem = (g0, g1, g2, g3, g4)
    ssem = (s0, s1, s2, s3, s4)
    cid = lax.axis_index("c")
    sid = lax.axis_index("s")
    wid = cid * NS + sid
    base = sid * RPT

    pltpu.sync_copy(ei_hbm.at[0, wid, 0], src_g.at[0])
    pltpu.sync_copy(ei_hbm.at[1, wid, 0], dst_g.at[0])
    pltpu.async_copy(ei_hbm.at[0, wid, 1], src_g.at[1], isem)
    pltpu.async_copy(ei_hbm.at[1, wid, 1], dst_g.at[1], isem)
    for sl in range(NBUF):
        pltpu.async_copy(g_hbm.at[src_g.at[0, sl]], rows[sl], gsem[sl])

    zeros16 = jnp.zeros((16,), jnp.float32)
    ZR = 32

    def zrow(i, carry):
        for jj in range(D // 16):
            zb[i, pl.ds(jj * 16, 16)] = zeros16
        return carry

    lax.fori_loop(0, ZR, zrow, 0)
    for t in range(RPT // ZR):
        pltpu.sync_copy(zb, acc_sh.at[pl.ds(base + t * ZR, ZR)])
    plsc.subcore_barrier()

    def ground(lr, g):
        p = g % 2
        pn = 1 - p
        last_g = g == NGRP - 1

        @pl.when(jnp.logical_and(lr == 1, jnp.logical_not(last_g)))
        def _():
            pltpu.async_copy(ei_hbm.at[0, wid, g + 1], src_g.at[pn], isem)
            pltpu.async_copy(ei_hbm.at[1, wid, g + 1], dst_g.at[pn], isem)

        @pl.when(jnp.logical_and(lr == RPG - 1, jnp.logical_not(last_g)))
        def _():
            pltpu.make_async_copy(ei_hbm.at[0, wid, g + 1], src_g.at[pn],
                                  isem).wait()
            pltpu.make_async_copy(ei_hbm.at[1, wid, g + 1], dst_g.at[pn],
                                  isem).wait()

        for sl in range(NBUF):
            lc = lr * NBUF + sl
            pltpu.make_async_copy(g_hbm.at[src_g.at[p, lc]], rows[sl],
                                  gsem[sl]).wait()
            pltpu.async_copy(rows[sl], acc_sh.at[dst_g.at[p, lc]],
                             ssem[sl], add=True)

        in_group = lr < RPG - 1
        to_next_group = jnp.logical_and(lr == RPG - 1,
                                        jnp.logical_not(last_g))
        for sl in range(NBUF):
            lc = lr * NBUF + sl

            @pl.when(jnp.logical_or(in_group, to_next_group))
            def _():
                pltpu.make_async_copy(rows[sl], acc_sh.at[dst_g.at[p, lc]],
                                      ssem[sl]).wait()

            @pl.when(in_group)
            def _():
                nlc = (lr + 1) * NBUF + sl
                pltpu.async_copy(g_hbm.at[src_g.at[p, nlc]], rows[sl],
                                 gsem[sl])

            @pl.when(to_next_group)
            def _():
                pltpu.async_copy(g_hbm.at[src_g.at[pn, sl]], rows[sl],
                                 gsem[sl])

    def group(g, carry):
        def rbody(lr, c2):
            ground(lr, g)
            return c2
        lax.fori_loop(0, RPG, rbody, 0)
        return carry

    lax.fori_loop(0, NGRP, group, 0)

    pl_last = (NGRP - 1) % 2
    for sl in range(NBUF):
        lc = (RPG - 1) * NBUF + sl
        pltpu.make_async_copy(rows[sl], acc_sh.at[dst_g.at[pl_last, lc]],
                              ssem[sl]).wait()

    plsc.subcore_barrier()
    sp = pl.ds(base, RPT)
    pltpu.sync_copy(acc_sh.at[sp], out_hbm.at[cid, sp])


@functools.lru_cache(maxsize=None)
def _sc_kernels():
    mesh = plsc.VectorSubcoreMesh(core_axis_name="c", subcore_axis_name="s",
                                  num_cores=NC, num_subcores=NS)
    deg_kernel = pl.kernel(
        _deg_body,
        out_type=jax.ShapeDtypeStruct((NC, NPAD), jnp.float32),
        mesh=mesh,
        scratch_types=[
            pltpu.VMEM((GRP, K), jnp.int32),
            pltpu.VMEM((K,), jnp.float32),
            pltpu.VMEM((PPT,), jnp.float32),
            pltpu.VMEM_SHARED((NPAD,), jnp.float32),
            pltpu.SemaphoreType.DMA,
        ],
    )
    edge_kernel = pl.kernel(
        _edge_body,
        out_type=jax.ShapeDtypeStruct((NC, NPAD, D), jnp.float32),
        mesh=mesh,
        scratch_types=(
            [pltpu.VMEM((2, GRP, K), jnp.int32)] * 2
            + [pltpu.VMEM((K, D), jnp.float32)] * NBUF
            + [pltpu.VMEM((32, D), jnp.float32)]
            + [pltpu.VMEM_SHARED((NPAD, D), jnp.float32)]
            + [pltpu.SemaphoreType.DMA] * (2 * NBUF + 1)
        ),
    )
    return deg_kernel, edge_kernel


def _dis(degp_ref):
    deg = 1.0 + degp_ref[:, 0:1] + degp_ref[:, 1:2]
    return lax.rsqrt(deg)


def _tc_scale_body(degp_ref, x_ref, w_ref, g_ref):
    h = lax.dot_general(x_ref[...], w_ref[...], (((1,), (0,)), ((), ())),
                        preferred_element_type=jnp.float32)
    g_ref[...] = h * _dis(degp_ref)


_tc_scale = pl.pallas_call(
    _tc_scale_body,
    out_shape=jax.ShapeDtypeStruct((N, D), jnp.float32),
)


def _tc_mid_body(s_ref, g_ref, degp_ref, b_ref, w_ref, g2_ref):
    dis = _dis(degp_ref)
    u = (s_ref[0, 0:N, :] + s_ref[1, 0:N, :] + g_ref[...]) * dis + b_ref[...]
    r = jnp.maximum(u, 0.0)
    h2 = lax.dot_general(r, w_ref[...], (((1,), (0,)), ((), ())),
                         preferred_element_type=jnp.float32)
    g2_ref[...] = h2 * dis


_tc_mid = pl.pallas_call(
    _tc_mid_body,
    out_shape=jax.ShapeDtypeStruct((N, D), jnp.float32),
)


def _tc_out_body(s_ref, g2_ref, degp_ref, b_ref, out_ref):
    out_ref[...] = (s_ref[0, 0:N, :] + s_ref[1, 0:N, :] + g2_ref[...]) \
        * _dis(degp_ref) + b_ref[...]


_tc_out = pl.pallas_call(
    _tc_out_body,
    out_shape=jax.ShapeDtypeStruct((N, D), jnp.float32),
)


def kernel(x, edge_index, W1, b1, W2, b2):
    ei = edge_index.astype(jnp.int32).reshape(2, NW, NGRP, GRP, K)
    _deg_kernel, _edge_kernel = _sc_kernels()

    degp = _deg_kernel(ei)
    degp_t = degp[:, :N].T

    g1 = _tc_scale(degp_t, x, W1)
    s1 = _edge_kernel(g1, ei)
    g2 = _tc_mid(s1, g1, degp_t, b1.reshape(1, D), W2)
    s2 = _edge_kernel(g2, ei)
    return _tc_out(s2, g2, degp_t, b2.reshape(1, D))


# speedup vs baseline: 1.0110x; 1.0054x over previous
import functools

import jax
import jax.numpy as jnp
from jax import lax
from jax.experimental import pallas as pl
from jax.experimental.pallas import tpu as pltpu
from jax.experimental.pallas import tpu_sc as plsc

N = 10000
E = 320000
D = 128
NC = 2
NS = 16
NW = NC * NS
EPW = E // NW
KD = 80
NCHUNKD = EPW // KD
GRPD = 25
NGRPD = NCHUNKD // GRPD
K = 40
NCHUNK = EPW // K
NBUF = 5
GRP = 25
NGRP = NCHUNK // GRP
RPG = GRP // NBUF
NPAD = 10240
RPT = NPAD // NS
PPT = NPAD // NS

def _deg_body(ei_hbm, out_hbm, idx_v, ones_v, stage_v, acc_sh, dsem):
    cid = lax.axis_index("c")
    sid = lax.axis_index("s")
    wid = cid * NS + sid

    zeros16 = jnp.zeros((16,), jnp.float32)
    for j in range(PPT // 16):
        stage_v[pl.ds(j * 16, 16)] = zeros16
    ones16 = jnp.ones((16,), jnp.float32)
    for off in (0, 16, 24):
        ones_v[pl.ds(off, 16)] = ones16

    pltpu.sync_copy(stage_v, acc_sh.at[pl.ds(sid * PPT, PPT)])
    plsc.subcore_barrier()

    def dgroup(g, carry):
        pltpu.sync_copy(ei_hbm.at[1, wid, g], idx_v)

        def fire(j, c2):
            pltpu.async_copy(ones_v, acc_sh.at[idx_v.at[j]], dsem, add=True)
            return c2

        lax.fori_loop(0, GRP, fire, 0)

        def drain(j, c2):
            pltpu.make_async_copy(ones_v, acc_sh.at[idx_v.at[j]],
                                  dsem).wait()
            return c2

        lax.fori_loop(0, GRP, drain, 0)
        return carry

    lax.fori_loop(0, NGRP, dgroup, 0)
    plsc.subcore_barrier()

    pltpu.sync_copy(acc_sh.at[pl.ds(sid * PPT, PPT)],
                    out_hbm.at[cid, pl.ds(sid * PPT, PPT)])


def _edge_body(g_hbm, ei_hbm, out_hbm,
               src_g, dst_g, r0, r1, r2, r3, r4, zb, acc_sh,
               g0, g1, g2, g3, g4, s0, s1, s2, s3, s4, isem):
    rows = (r0, r1, r2, r3, r4)
    gsem = (g0, g1, g2, g3, g4)
    ssem = (s0, s1, s2, s3, s4)
    cid = lax.axis_index("c")
    sid = lax.axis_index("s")
    wid = cid * NS + sid
    base = sid * RPT

    pltpu.sync_copy(ei_hbm.at[0, wid, 0], src_g.at[0])
    pltpu.sync_copy(ei_hbm.at[1, wid, 0], dst_g.at[0])
    pltpu.async_copy(ei_hbm.at[0, wid, 1], src_g.at[1], isem)
    pltpu.async_copy(ei_hbm.at[1, wid, 1], dst_g.at[1], isem)
    for sl in range(NBUF):
        pltpu.async_copy(g_hbm.at[src_g.at[0, sl]], rows[sl], gsem[sl])

    zeros16 = jnp.zeros((16,), jnp.float32)
    ZR = 32

    def zrow(i, carry):
        for jj in range(D // 16):
            zb[i, pl.ds(jj * 16, 16)] = zeros16
        return carry

    lax.fori_loop(0, ZR, zrow, 0)
    for t in range(RPT // ZR):
        pltpu.sync_copy(zb, acc_sh.at[pl.ds(base + t * ZR, ZR)])
    plsc.subcore_barrier()

    def ground(lr, g):
        p = g % 2
        pn = 1 - p
        last_g = g == NGRP - 1

        @pl.when(jnp.logical_and(lr == 1, jnp.logical_not(last_g)))
        def _():
            pltpu.async_copy(ei_hbm.at[0, wid, g + 1], src_g.at[pn], isem)
            pltpu.async_copy(ei_hbm.at[1, wid, g + 1], dst_g.at[pn], isem)

        @pl.when(jnp.logical_and(lr == RPG - 1, jnp.logical_not(last_g)))
        def _():
            pltpu.make_async_copy(ei_hbm.at[0, wid, g + 1], src_g.at[pn],
                                  isem).wait()
            pltpu.make_async_copy(ei_hbm.at[1, wid, g + 1], dst_g.at[pn],
                                  isem).wait()

        for sl in range(NBUF):
            lc = lr * NBUF + sl
            pltpu.make_async_copy(g_hbm.at[src_g.at[p, lc]], rows[sl],
                                  gsem[sl]).wait()
            pltpu.async_copy(rows[sl], acc_sh.at[dst_g.at[p, lc]],
                             ssem[sl], add=True)

        in_group = lr < RPG - 1
        to_next_group = jnp.logical_and(lr == RPG - 1,
                                        jnp.logical_not(last_g))
        for sl in range(NBUF):
            lc = lr * NBUF + sl

            @pl.when(jnp.logical_or(in_group, to_next_group))
            def _():
                pltpu.make_async_copy(rows[sl], acc_sh.at[dst_g.at[p, lc]],
                                      ssem[sl]).wait()

            @pl.when(in_group)
            def _():
                nlc = (lr + 1) * NBUF + sl
                pltpu.async_copy(g_hbm.at[src_g.at[p, nlc]], rows[sl],
                                 gsem[sl])

            @pl.when(to_next_group)
            def _():
                pltpu.async_copy(g_hbm.at[src_g.at[pn, sl]], rows[sl],
                                 gsem[sl])

    def group(g, carry):
        def rbody(lr, c2):
            ground(lr, g)
            return c2
        lax.fori_loop(0, RPG, rbody, 0)
        return carry

    lax.fori_loop(0, NGRP, group, 0)

    pl_last = (NGRP - 1) % 2
    for sl in range(NBUF):
        lc = (RPG - 1) * NBUF + sl
        pltpu.make_async_copy(rows[sl], acc_sh.at[dst_g.at[pl_last, lc]],
                              ssem[sl]).wait()

    plsc.subcore_barrier()
    sp = pl.ds(base, RPT)
    pltpu.sync_copy(acc_sh.at[sp], out_hbm.at[cid, sp])


@functools.lru_cache(maxsize=None)
def _sc_kernels():
    mesh = plsc.VectorSubcoreMesh(core_axis_name="c", subcore_axis_name="s",
                                  num_cores=NC, num_subcores=NS)
    deg_kernel = pl.kernel(
        _deg_body,
        out_type=jax.ShapeDtypeStruct((NC, NPAD), jnp.float32),
        mesh=mesh,
        scratch_types=[
            pltpu.VMEM((GRP, K), jnp.int32),
            pltpu.VMEM((K,), jnp.float32),
            pltpu.VMEM((PPT,), jnp.float32),
            pltpu.VMEM_SHARED((NPAD,), jnp.float32),
            pltpu.SemaphoreType.DMA,
        ],
    )
    edge_kernel = pl.kernel(
        _edge_body,
        out_type=jax.ShapeDtypeStruct((NC, NPAD, D), jnp.float32),
        mesh=mesh,
        scratch_types=(
            [pltpu.VMEM((2, GRP, K), jnp.int32)] * 2
            + [pltpu.VMEM((K, D), jnp.float32)] * NBUF
            + [pltpu.VMEM((32, D), jnp.float32)]
            + [pltpu.VMEM_SHARED((NPAD, D), jnp.float32)]
            + [pltpu.SemaphoreType.DMA] * (2 * NBUF + 1)
        ),
    )
    return deg_kernel, edge_kernel


def _dis(degp_ref):
    deg = 1.0 + degp_ref[:, 0:1] + degp_ref[:, 1:2]
    return lax.rsqrt(deg)


BR = 2000
NBL = N // BR


def _tc_scale_body(degp_ref, x_ref, w_ref, g_ref):
    h = lax.dot_general(x_ref[...], w_ref[...], (((1,), (0,)), ((), ())),
                        preferred_element_type=jnp.float32)
    g_ref[...] = h * _dis(degp_ref)


_tc_scale = pl.pallas_call(
    _tc_scale_body,
    grid=(NBL,),
    in_specs=[
        pl.BlockSpec((BR, NC), lambda i: (i, 0)),
        pl.BlockSpec((BR, D), lambda i: (i, 0)),
        pl.BlockSpec((D, D), lambda i: (0, 0)),
    ],
    out_specs=pl.BlockSpec((BR, D), lambda i: (i, 0)),
    out_shape=jax.ShapeDtypeStruct((N, D), jnp.float32),
)


def _tc_mid_body(s_ref, g_ref, degp_ref, b_ref, w_ref, g2_ref):
    dis = _dis(degp_ref)
    u = (s_ref[0] + s_ref[1] + g_ref[...]) * dis + b_ref[...]
    r = jnp.maximum(u, 0.0)
    h2 = lax.dot_general(r, w_ref[...], (((1,), (0,)), ((), ())),
                         preferred_element_type=jnp.float32)
    g2_ref[...] = h2 * dis


_tc_mid = pl.pallas_call(
    _tc_mid_body,
    grid=(NBL,),
    in_specs=[
        pl.BlockSpec((NC, BR, D), lambda i: (0, i, 0)),
        pl.BlockSpec((BR, D), lambda i: (i, 0)),
        pl.BlockSpec((BR, NC), lambda i: (i, 0)),
        pl.BlockSpec((1, D), lambda i: (0, 0)),
        pl.BlockSpec((D, D), lambda i: (0, 0)),
    ],
    out_specs=pl.BlockSpec((BR, D), lambda i: (i, 0)),
    out_shape=jax.ShapeDtypeStruct((N, D), jnp.float32),
)


def _tc_out_body(s_ref, g2_ref, degp_ref, b_ref, out_ref):
    out_ref[...] = (s_ref[0] + s_ref[1] + g2_ref[...]) \
        * _dis(degp_ref) + b_ref[...]


_tc_out = pl.pallas_call(
    _tc_out_body,
    grid=(NBL,),
    in_specs=[
        pl.BlockSpec((NC, BR, D), lambda i: (0, i, 0)),
        pl.BlockSpec((BR, D), lambda i: (i, 0)),
        pl.BlockSpec((BR, NC), lambda i: (i, 0)),
        pl.BlockSpec((1, D), lambda i: (0, 0)),
    ],
    out_specs=pl.BlockSpec((BR, D), lambda i: (i, 0)),
    out_shape=jax.ShapeDtypeStruct((N, D), jnp.float32),
)


def kernel(x, edge_index, W1, b1, W2, b2):
    ei = edge_index.astype(jnp.int32).reshape(2, NW, NGRP, GRP, K)
    _deg_kernel, _edge_kernel = _sc_kernels()

    degp = _deg_kernel(ei)
    degp_t = degp[:, :N].T

    g1 = _tc_scale(degp_t, x, W1)
    s1 = _edge_kernel(g1, ei)
    g2 = _tc_mid(s1, g1, degp_t, b1.reshape(1, D), W2)
    s2 = _edge_kernel(g2, ei)
    return _tc_out(s2, g2, degp_t, b2.reshape(1, D))


# speedup vs baseline: 1.0119x; 1.0009x over previous
import functools

import jax
import jax.numpy as jnp
from jax import lax
from jax.experimental import pallas as pl
from jax.experimental.pallas import tpu as pltpu
from jax.experimental.pallas import tpu_sc as plsc

N = 10000
E = 320000
D = 128
NC = 2
NS = 16
NW = NC * NS
EPW = E // NW
K = 40
NCHUNK = EPW // K
NBUF = 5
GRP = 25
NGRP = NCHUNK // GRP
RPG = GRP // NBUF
NPAD = 10240
RPT = NPAD // NS
PPT = NPAD // NS

def _deg_body(ei_hbm, out_hbm, idx_v, ones_v, stage_v, acc_sh, dsem):
    cid = lax.axis_index("c")
    sid = lax.axis_index("s")
    wid = cid * NS + sid

    zeros16 = jnp.zeros((16,), jnp.float32)
    for j in range(PPT // 16):
        stage_v[pl.ds(j * 16, 16)] = zeros16
    ones16 = jnp.ones((16,), jnp.float32)
    for off in (0, 16, 24):
        ones_v[pl.ds(off, 16)] = ones16

    pltpu.sync_copy(stage_v, acc_sh.at[pl.ds(sid * PPT, PPT)])
    plsc.subcore_barrier()

    def dgroup(g, carry):
        pltpu.sync_copy(ei_hbm.at[1, wid, g], idx_v)

        def fire(j, c2):
            pltpu.async_copy(ones_v, acc_sh.at[idx_v.at[j]], dsem, add=True)
            return c2

        lax.fori_loop(0, GRP, fire, 0)

        def drain(j, c2):
            pltpu.make_async_copy(ones_v, acc_sh.at[idx_v.at[j]],
                                  dsem).wait()
            return c2

        lax.fori_loop(0, GRP, drain, 0)
        return carry

    lax.fori_loop(0, NGRP, dgroup, 0)
    plsc.subcore_barrier()

    pltpu.sync_copy(acc_sh.at[pl.ds(sid * PPT, PPT)],
                    out_hbm.at[cid, pl.ds(sid * PPT, PPT)])


def _edge_body(g_hbm, ei_hbm, out_hbm,
               src_g, dst_g, r0, r1, r2, r3, r4, zb, acc_sh,
               g0, g1, g2, g3, g4, s0, s1, s2, s3, s4, isem):
    rows = (r0, r1, r2, r3, r4)
    gsem = (g0, g1, g2, g3, g4)
    ssem = (s0, s1, s2, s3, s4)
    cid = lax.axis_index("c")
    sid = lax.axis_index("s")
    wid = cid * NS + sid
    base = sid * RPT

    pltpu.sync_copy(ei_hbm.at[0, wid, 0], src_g.at[0])
    pltpu.sync_copy(ei_hbm.at[1, wid, 0], dst_g.at[0])
    pltpu.async_copy(ei_hbm.at[0, wid, 1], src_g.at[1], isem)
    pltpu.async_copy(ei_hbm.at[1, wid, 1], dst_g.at[1], isem)
    for sl in range(NBUF):
        pltpu.async_copy(g_hbm.at[src_g.at[0, sl]], rows[sl], gsem[sl])

    zeros16 = jnp.zeros((16,), jnp.float32)
    ZR = 32

    def zrow(i, carry):
        for jj in range(D // 16):
            zb[i, pl.ds(jj * 16, 16)] = zeros16
        return carry

    lax.fori_loop(0, ZR, zrow, 0)
    for t in range(RPT // ZR):
        pltpu.sync_copy(zb, acc_sh.at[pl.ds(base + t * ZR, ZR)])
    plsc.subcore_barrier()

    def ground(lr, g):
        p = g % 2
        pn = 1 - p
        last_g = g == NGRP - 1

        @pl.when(jnp.logical_and(lr == 1, jnp.logical_not(last_g)))
        def _():
            pltpu.async_copy(ei_hbm.at[0, wid, g + 1], src_g.at[pn], isem)
            pltpu.async_copy(ei_hbm.at[1, wid, g + 1], dst_g.at[pn], isem)

        @pl.when(jnp.logical_and(lr == RPG - 1, jnp.logical_not(last_g)))
        def _():
            pltpu.make_async_copy(ei_hbm.at[0, wid, g + 1], src_g.at[pn],
                                  isem).wait()
            pltpu.make_async_copy(ei_hbm.at[1, wid, g + 1], dst_g.at[pn],
                                  isem).wait()

        for sl in range(NBUF):
            lc = lr * NBUF + sl
            pltpu.make_async_copy(g_hbm.at[src_g.at[p, lc]], rows[sl],
                                  gsem[sl]).wait()
            pltpu.async_copy(rows[sl], acc_sh.at[dst_g.at[p, lc]],
                             ssem[sl], add=True)

        in_group = lr < RPG - 1
        to_next_group = jnp.logical_and(lr == RPG - 1,
                                        jnp.logical_not(last_g))
        for sl in range(NBUF):
            lc = lr * NBUF + sl

            @pl.when(jnp.logical_or(in_group, to_next_group))
            def _():
                pltpu.make_async_copy(rows[sl], acc_sh.at[dst_g.at[p, lc]],
                                      ssem[sl]).wait()

            @pl.when(in_group)
            def _():
                nlc = (lr + 1) * NBUF + sl
                pltpu.async_copy(g_hbm.at[src_g.at[p, nlc]], rows[sl],
                                 gsem[sl])

            @pl.when(to_next_group)
            def _():
                pltpu.async_copy(g_hbm.at[src_g.at[pn, sl]], rows[sl],
                                 gsem[sl])

    def group(g, carry):
        def rbody(lr, c2):
            ground(lr, g)
            return c2
        lax.fori_loop(0, RPG, rbody, 0)
        return carry

    lax.fori_loop(0, NGRP, group, 0)

    pl_last = (NGRP - 1) % 2
    for sl in range(NBUF):
        lc = (RPG - 1) * NBUF + sl
        pltpu.make_async_copy(rows[sl], acc_sh.at[dst_g.at[pl_last, lc]],
                              ssem[sl]).wait()

    plsc.subcore_barrier()
    sp = pl.ds(base, RPT)
    pltpu.sync_copy(acc_sh.at[sp], out_hbm.at[cid, sp])


@functools.lru_cache(maxsize=None)
def _sc_kernels():
    mesh = plsc.VectorSubcoreMesh(core_axis_name="c", subcore_axis_name="s",
                                  num_cores=NC, num_subcores=NS)
    deg_kernel = pl.kernel(
        _deg_body,
        out_type=jax.ShapeDtypeStruct((NC, NPAD), jnp.float32),
        mesh=mesh,
        scratch_types=[
            pltpu.VMEM((GRP, K), jnp.int32),
            pltpu.VMEM((K,), jnp.float32),
            pltpu.VMEM((PPT,), jnp.float32),
            pltpu.VMEM_SHARED((NPAD,), jnp.float32),
            pltpu.SemaphoreType.DMA,
        ],
    )
    edge_kernel = pl.kernel(
        _edge_body,
        out_type=jax.ShapeDtypeStruct((NC, NPAD, D), jnp.float32),
        mesh=mesh,
        scratch_types=(
            [pltpu.VMEM((2, GRP, K), jnp.int32)] * 2
            + [pltpu.VMEM((K, D), jnp.float32)] * NBUF
            + [pltpu.VMEM((32, D), jnp.float32)]
            + [pltpu.VMEM_SHARED((NPAD, D), jnp.float32)]
            + [pltpu.SemaphoreType.DMA] * (2 * NBUF + 1)
        ),
    )
    return deg_kernel, edge_kernel


def _dis(degp_ref):
    deg = 1.0 + degp_ref[:, 0:1] + degp_ref[:, 1:2]
    return lax.rsqrt(deg)


BR = 2000
NBL = N // BR


def _tc_scale_body(degp_ref, x_ref, w_ref, g_ref):
    h = lax.dot_general(x_ref[...], w_ref[...], (((1,), (0,)), ((), ())),
                        preferred_element_type=jnp.float32)
    g_ref[...] = h * _dis(degp_ref)


_tc_scale = pl.pallas_call(
    _tc_scale_body,
    grid=(NBL,),
    in_specs=[
        pl.BlockSpec((BR, NC), lambda i: (i, 0)),
        pl.BlockSpec((BR, D), lambda i: (i, 0)),
        pl.BlockSpec((D, D), lambda i: (0, 0)),
    ],
    out_specs=pl.BlockSpec((BR, D), lambda i: (i, 0)),
    out_shape=jax.ShapeDtypeStruct((N, D), jnp.float32),
)


def _tc_mid_body(s_ref, g_ref, degp_ref, b_ref, w_ref, g2_ref):
    dis = _dis(degp_ref)
    u = (s_ref[0] + s_ref[1] + g_ref[...]) * dis + b_ref[...]
    r = jnp.maximum(u, 0.0)
    h2 = lax.dot_general(r, w_ref[...], (((1,), (0,)), ((), ())),
                         preferred_element_type=jnp.float32)
    g2_ref[...] = h2 * dis


_tc_mid = pl.pallas_call(
    _tc_mid_body,
    grid=(NBL,),
    in_specs=[
        pl.BlockSpec((NC, BR, D), lambda i: (0, i, 0)),
        pl.BlockSpec((BR, D), lambda i: (i, 0)),
        pl.BlockSpec((BR, NC), lambda i: (i, 0)),
        pl.BlockSpec((1, D), lambda i: (0, 0)),
        pl.BlockSpec((D, D), lambda i: (0, 0)),
    ],
    out_specs=pl.BlockSpec((BR, D), lambda i: (i, 0)),
    out_shape=jax.ShapeDtypeStruct((N, D), jnp.float32),
)


def _tc_out_body(s_ref, g2_ref, degp_ref, b_ref, out_ref):
    out_ref[...] = (s_ref[0] + s_ref[1] + g2_ref[...]) \
        * _dis(degp_ref) + b_ref[...]


_tc_out = pl.pallas_call(
    _tc_out_body,
    grid=(NBL,),
    in_specs=[
        pl.BlockSpec((NC, BR, D), lambda i: (0, i, 0)),
        pl.BlockSpec((BR, D), lambda i: (i, 0)),
        pl.BlockSpec((BR, NC), lambda i: (i, 0)),
        pl.BlockSpec((1, D), lambda i: (0, 0)),
    ],
    out_specs=pl.BlockSpec((BR, D), lambda i: (i, 0)),
    out_shape=jax.ShapeDtypeStruct((N, D), jnp.float32),
)


def kernel(x, edge_index, W1, b1, W2, b2):
    ei = edge_index.astype(jnp.int32).reshape(2, NW, NGRP, GRP, K)
    _deg_kernel, _edge_kernel = _sc_kernels()

    degp = _deg_kernel(ei)
    degp_t = degp[:, :N].T

    g1 = _tc_scale(degp_t, x, W1)
    s1 = _edge_kernel(g1, ei)
    g2 = _tc_mid(s1, g1, degp_t, b1.reshape(1, D), W2)
    s2 = _edge_kernel(g2, ei)
    return _tc_out(s2, g2, degp_t, b2.reshape(1, D))
